# baseline (speedup 1.0000x reference)
import functools

import jax
import jax.numpy as jnp
from jax import lax
from jax.experimental import pallas as pl
from jax.experimental.pallas import tpu as pltpu, tpu_sc as plsc

B = 16384
D = 128
VOCABS = [4, 2, 2, 5, 3, 4, 4]
NT = len(VOCABS)
NROWS = sum(VOCABS)
RPAD = 32
OFFS = [0]
for _v in VOCABS[:-1]:
    OFFS.append(OFFS[-1] + _v)

NC = 2
NS = 16
NW = NC * NS
BPW = B // NW
LANES = 16
NCHUNK = BPW // LANES
NGRP = RPAD // LANES
DUNROLL = 4
MUNROLL = 2


def _tanh16(x):
    ax = jnp.abs(x)
    e = jnp.exp(-2.0 * ax)
    return jnp.sign(x) * ((1.0 - e) / (1.0 + e))


def _sc_body(x_hbm, e_hbm, w_hbm, out_hbm, xv, ev, wv, sv, outv, sem):
    wid = lax.axis_index("s") * NC + lax.axis_index("c")
    base = wid * BPW

    copies = [
        pltpu.async_copy(e_hbm, ev, sem),
        pltpu.async_copy(w_hbm, wv, sem),
        pltpu.async_copy(x_hbm.at[:, pl.ds(base, BPW)], xv, sem),
    ]
    for c in copies:
        c.wait()

    lane = lax.iota(jnp.int32, LANES)
    ebase, wbase = [], []
    for g in range(NGRP):
        row = jnp.minimum(lane + g * LANES, NROWS - 1)
        tbl = jnp.zeros((LANES,), jnp.int32)
        for i in range(1, NT):
            tbl = tbl + (row >= OFFS[i]).astype(jnp.int32)
        ebase.append(row * D)
        wbase.append(tbl * D)

    def pre_body(it, accs):
        new = list(accs)
        for u in range(DUNROLL):
            d = it * DUNROLL + u
            for g in range(NGRP):
                evec = plsc.load_gather(ev, [ebase[g] + d])
                pvec = plsc.load_gather(wv, [wbase[g] + d])
                new[g] = new[g] + _tanh16(evec) * pvec
        return tuple(new)

    zero = jnp.zeros((LANES,), jnp.float32)
    accs = lax.fori_loop(0, D // DUNROLL, pre_body, (zero,) * NGRP)
    for g in range(NGRP):
        sv[pl.ds(g * LANES, LANES)] = accs[g]

    def chunk_body(j, carry):
        for u in range(MUNROLL):
            off = (j * MUNROLL + u) * LANES
            acc = None
            for i in range(NT):
                idx = xv[i, pl.ds(off, LANES)] + OFFS[i]
                g = plsc.load_gather(sv, [idx])
                acc = g if acc is None else acc + g
            outv[pl.ds(off, LANES)] = acc
        return carry

    lax.fori_loop(0, NCHUNK // MUNROLL, chunk_body, 0)

    pltpu.sync_copy(outv, out_hbm.at[pl.ds(base, BPW)])


@jax.jit
def _run(x, epk, w):
    mesh = plsc.VectorSubcoreMesh(core_axis_name="c", subcore_axis_name="s")
    f = functools.partial(
        pl.kernel,
        mesh=mesh,
        out_type=jax.ShapeDtypeStruct((B,), jnp.float32),
        scratch_types=[
            pltpu.VMEM((NT, BPW), jnp.int32),
            pltpu.VMEM((NROWS * D,), jnp.float32),
            pltpu.VMEM((NT * D,), jnp.float32),
            pltpu.VMEM((RPAD,), jnp.float32),
            pltpu.VMEM((BPW,), jnp.float32),
            pltpu.SemaphoreType.DMA,
        ],
        compiler_params=pltpu.CompilerParams(needs_layout_passes=False),
    )(_sc_body)
    return f(x, epk, w)


def kernel(input, E1, E2, E3, E4, E5, E6, E7, W):
    epk = jnp.concatenate(
        [E1, E2, E3, E4, E5, E6, E7], axis=0
    ).reshape(-1)
    out = _run(input, epk, W.reshape(-1))
    return out.reshape(B, 1)


# speedup vs baseline: 1.1678x; 1.1678x over previous
import functools

import jax
import jax.numpy as jnp
from jax import lax
from jax.experimental import pallas as pl
from jax.experimental.pallas import tpu as pltpu, tpu_sc as plsc

B = 16384
D = 128
VOCABS = [4, 2, 2, 5, 3, 4, 4]
NT = len(VOCABS)
NROWS = sum(VOCABS)
OFFS = [0]
for _v in VOCABS[:-1]:
    OFFS.append(OFFS[-1] + _v)

NC = 2
NS = 16
NW = NC * NS
BPW = B // NW
LANES = 16
NCHUNK = BPW // LANES
DCHUNK = D // LANES
MUNROLL = 2
PSTRIDE = 24


def _tanh16(x):
    ax = jnp.abs(x)
    e = jnp.exp(-2.0 * ax)
    return jnp.sign(x) * ((1.0 - e) / (1.0 + e))


def _sc_body(x_hbm, e_hbms, w_hbm, out_hbm,
             xv, ev, wv, pv, sv, outv, semw, semx):
    wid = lax.axis_index("s") * NC + lax.axis_index("c")
    base = wid * BPW

    xcopy = pltpu.async_copy(x_hbm.at[:, pl.ds(base, BPW)], xv, semx)
    wcopies = [pltpu.async_copy(w_hbm, wv, semw)]
    for i in range(NT):
        wcopies.append(
            pltpu.async_copy(
                e_hbms[i], ev.at[pl.ds(OFFS[i] * D, VOCABS[i] * D)], semw
            )
        )
    for c in wcopies:
        c.wait()

    def row_body(r, carry):
        t = jnp.int32(0)
        for i in range(1, NT):
            t = t + (r >= OFFS[i]).astype(jnp.int32)
        eoff = r * D
        woff = t * D
        acc = None
        for k in range(DCHUNK):
            e = ev[pl.ds(eoff + k * LANES, LANES)]
            w = wv[pl.ds(woff + k * LANES, LANES)]
            term = _tanh16(e) * w
            acc = term if acc is None else acc + term
        pv[pl.ds(r * PSTRIDE, LANES)] = acc
        return carry

    lax.fori_loop(0, NROWS, row_body, 0)

    lane = lax.iota(jnp.int32, LANES)
    rowb = lane * PSTRIDE
    for g in range(2):
        svec = None
        for l in range(LANES):
            gv = plsc.load_gather(pv, [rowb + (g * LANES * PSTRIDE + l)])
            svec = gv if svec is None else svec + gv
        sv[pl.ds(g * LANES, LANES)] = svec

    xcopy.wait()

    def chunk_body(j, carry):
        for u in range(MUNROLL):
            off = (j * MUNROLL + u) * LANES
            acc = None
            for i in range(NT):
                idx = xv[i, pl.ds(off, LANES)] + OFFS[i]
                g = plsc.load_gather(sv, [idx])
                acc = g if acc is None else acc + g
            outv[pl.ds(off, LANES)] = acc
        return carry

    lax.fori_loop(0, NCHUNK // MUNROLL, chunk_body, 0)

    pltpu.sync_copy(outv, out_hbm.at[pl.ds(base, BPW)])


@jax.jit
def _run(x, e1, e2, e3, e4, e5, e6, e7, w):
    mesh = plsc.VectorSubcoreMesh(core_axis_name="c", subcore_axis_name="s")

    def body(x_r, e1_r, e2_r, e3_r, e4_r, e5_r, e6_r, e7_r, w_r, out_r,
             xv, ev, wv, pv, sv, outv, semw, semx):
        _sc_body(x_r, (e1_r, e2_r, e3_r, e4_r, e5_r, e6_r, e7_r), w_r,
                 out_r, xv, ev, wv, pv, sv, outv, semw, semx)

    f = functools.partial(
        pl.kernel,
        mesh=mesh,
        out_type=jax.ShapeDtypeStruct((B,), jnp.float32),
        scratch_types=[
            pltpu.VMEM((NT, BPW), jnp.int32),
            pltpu.VMEM((NROWS * D,), jnp.float32),
            pltpu.VMEM((NT * D,), jnp.float32),
            pltpu.VMEM((2 * LANES * PSTRIDE,), jnp.float32),
            pltpu.VMEM((2 * LANES,), jnp.float32),
            pltpu.VMEM((BPW,), jnp.float32),
            pltpu.SemaphoreType.DMA,
            pltpu.SemaphoreType.DMA,
        ],
        compiler_params=pltpu.CompilerParams(needs_layout_passes=False),
    )(body)
    return f(x, e1, e2, e3, e4, e5, e6, e7, w)


def kernel(input, E1, E2, E3, E4, E5, E6, E7, W):
    es = [e.reshape(-1) for e in (E1, E2, E3, E4, E5, E6, E7)]
    out = _run(input, *es, W.reshape(-1))
    return out.reshape(B, 1)
